# Initial kernel scaffold; baseline (speedup 1.0000x reference)
import jax
import jax.numpy as jnp
from jax.experimental import pallas as pl


def kernel(key_soft, W):
    raise NotImplementedError("write your pallas kernel here")


# speedup vs baseline: 1.5410x; 1.5410x over previous
import functools

import jax
import jax.numpy as jnp
from jax.experimental import pallas as pl

_N_E = 8192
_E = 256
_B = 16
_T = 256
_LC = 0.2

_CB = 512
_NPAD = 8704
_NBLK = _NPAD // _CB


def _dist_block_kernel(ks_ref, w_ref, d_ref):
    ks = ks_ref[...]
    w = w_ref[...]
    s1 = jnp.sum(ks * ks, axis=1, keepdims=True)
    s2 = jnp.sum(w * w, axis=1)
    mm = jax.lax.dot_general(
        ks, w, (((1,), (1,)), ((), ())),
        preferred_element_type=jnp.float32)
    d_ref[...] = s1 + s2[None, :] - 2.0 * mm


def _dist_matrix(ksf, w_pad):
    bt = ksf.shape[0]
    return pl.pallas_call(
        _dist_block_kernel,
        grid=(_NBLK,),
        in_specs=[
            pl.BlockSpec((bt, _E), lambda i: (0, 0)),
            pl.BlockSpec((_CB, _E), lambda i: (i, 0)),
        ],
        out_specs=pl.BlockSpec((bt, _CB), lambda i: (0, i)),
        out_shape=jax.ShapeDtypeStruct((bt, _NPAD), jnp.float32),
    )(ksf, w_pad)


def _compute_indices_scan(d3, n_e):
    enc0 = jnp.clip(jnp.argmin(d3[:, 0, :], axis=1), 0, n_e - 1)

    def step(ind_here, d_t):
        d_here = jnp.take_along_axis(d_t, ind_here[:, None], axis=1)[:, 0]
        ind_next = jnp.clip(ind_here + 1, 0, n_e - 1)
        d_next = jnp.take_along_axis(d_t, ind_next[:, None], axis=1)[:, 0]
        ind_new = jnp.where(d_here <= d_next, ind_here, ind_next)
        return ind_new, ind_new

    _, inds = jax.lax.scan(step, enc0, jnp.transpose(d3[:, 1:, :], (1, 0, 2)))
    return jnp.concatenate([enc0[:, None], jnp.transpose(inds, (1, 0))], axis=1)


def kernel(key_soft, W):
    Bx, Tx, e_dim = key_soft.shape
    n_e = W.shape[0] - 1

    ksf = key_soft.reshape(-1, e_dim)
    w_pad = jnp.concatenate(
        [W, jnp.full((_NPAD - (n_e + 1), e_dim), 30000.0, jnp.float32)], axis=0)

    d = _dist_matrix(ksf, w_pad)[:, : n_e + 1]
    min_indices = jnp.argmin(d, axis=1)
    d3 = d.reshape(Bx, Tx, n_e + 1)
    encoding_indices = _compute_indices_scan(d3, n_e)

    eif = encoding_indices.reshape(-1)
    key_hard_here = jnp.take(W, eif, axis=0).reshape(key_soft.shape)
    key_hard_next = jnp.take(W, jnp.clip(eif + 1, 0, n_e - 1), axis=0).reshape(key_soft.shape)
    key_min = jnp.take(W, min_indices, axis=0).reshape(key_soft.shape)

    lc = _LC
    loss_here_base = jnp.sum((key_soft - key_hard_here) ** 2, axis=-1) * lc + jnp.sum((key_soft - key_hard_here) ** 2, axis=-1)
    loss_next_base = jnp.sum((key_soft - key_hard_next) ** 2, axis=-1) * lc + jnp.sum((key_soft - key_hard_next) ** 2, axis=-1)
    loss_min_indices = jnp.sum((key_soft - key_min) ** 2, axis=-1) + jnp.sum((key_soft - key_min) ** 2, axis=-1) * lc
    loss_min_here = jnp.where(loss_min_indices < loss_here_base, loss_min_indices, 0.0)
    loss_min_next = jnp.where(loss_min_indices < loss_next_base, loss_min_indices, 0.0)
    loss_here = loss_here_base - loss_min_here
    loss_next = loss_next_base - loss_min_next

    key_hard = key_hard_here

    min_i = jnp.min(encoding_indices, axis=1)
    max_i = jnp.max(encoding_indices, axis=1)
    v = jnp.max(max_i - min_i)
    return (key_hard, encoding_indices, v, loss_here, loss_next)


# speedup vs baseline: 7.0238x; 4.5580x over previous
import functools

import jax
import jax.numpy as jnp
from jax.experimental import pallas as pl
from jax.experimental.pallas import tpu as pltpu

_N_E = 8192
_E = 256
_B = 16
_T = 256
_LC = 0.2

_CB = 512
_NPAD = 8704
_NBLK = _NPAD // _CB
_W = 512


def _argmin_kernel(ks_ref, w_ref, mv_ref, mi_ref, rv_ref, ri_ref):
    i = pl.program_id(0)
    ks = ks_ref[...]
    w = w_ref[...]
    s1 = jnp.sum(ks * ks, axis=1, keepdims=True)
    s2 = jnp.sum(w * w, axis=1)
    mm = jax.lax.dot_general(ks, w, (((1,), (1,)), ((), ())),
                             preferred_element_type=jnp.float32)
    d = s1 + s2[None, :] - 2.0 * mm

    bval = jnp.min(d, axis=1, keepdims=True)
    iot = jax.lax.broadcasted_iota(jnp.int32, d.shape, 1)
    bidx = jnp.min(jnp.where(d == bval, iot, 2 ** 30), axis=1,
                   keepdims=True) + i * _CB

    @pl.when(i == 0)
    def _():
        rv_ref[...] = bval
        ri_ref[...] = bidx

    @pl.when(i > 0)
    def _():
        upd = bval < rv_ref[...]
        rv_ref[...] = jnp.where(upd, bval, rv_ref[...])
        ri_ref[...] = jnp.where(upd, bidx, ri_ref[...])

    @pl.when(i == _NBLK - 1)
    def _():
        mv_ref[...] = rv_ref[...]
        mi_ref[...] = ri_ref[...]


def _argmin_pass(ksf, w_pad):
    bt = ksf.shape[0]
    return pl.pallas_call(
        _argmin_kernel,
        grid=(_NBLK,),
        in_specs=[
            pl.BlockSpec((bt, _E), lambda i: (0, 0)),
            pl.BlockSpec((_CB, _E), lambda i: (i, 0)),
        ],
        out_specs=[
            pl.BlockSpec((bt, 1), lambda i: (0, 0)),
            pl.BlockSpec((bt, 1), lambda i: (0, 0)),
        ],
        out_shape=[
            jax.ShapeDtypeStruct((bt, 1), jnp.float32),
            jax.ShapeDtypeStruct((bt, 1), jnp.int32),
        ],
        scratch_shapes=[
            pltpu.VMEM((bt, 1), jnp.float32),
            pltpu.VMEM((bt, 1), jnp.int32),
        ],
    )(ksf, w_pad)


def _walk_kernel(s_ref, j0_ref, ks_ref, w_ref,
                 enc_ref, v_ref, kh_ref, kn_ref,
                 dw_s, adv_s, p_s):
    iot_j = jax.lax.broadcasted_iota(jnp.int32, (_B, _W), 1)

    for b in range(_B):
        s = pl.multiple_of(s_ref[b], 256)
        wwin = w_ref[pl.ds(s, _W), :]
        ks_b = ks_ref[b]
        s1 = jnp.sum(ks_b * ks_b, axis=1, keepdims=True)
        s2 = jnp.sum(wwin * wwin, axis=1)
        mm = jax.lax.dot_general(ks_b, wwin, (((1,), (1,)), ((), ())),
                                 preferred_element_type=jnp.float32)
        dw = s1 + s2[None, :] - 2.0 * mm
        dw_s[:, b, :] = dw
        dnext = jnp.concatenate([dw[:, 1:], dw[:, :1]], axis=1)
        limit = _N_E - 1 - s_ref[b]
        adv = (dnext < dw) & (jax.lax.broadcasted_iota(jnp.int32, dw.shape, 1)
                              < limit)
        adv_s[:, b, :] = adv.astype(jnp.float32)

    iot_1w = jax.lax.broadcasted_iota(jnp.int32, (1, _W), 1)
    p0 = jnp.concatenate(
        [(iot_1w == j0_ref[b]).astype(jnp.float32) for b in range(_B)],
        axis=0)
    j0 = jnp.concatenate(
        [jnp.zeros((1, 1), jnp.int32) + j0_ref[b] for b in range(_B)], axis=0)
    lim_row = jnp.concatenate(
        [jnp.zeros((1, 1), jnp.int32) + (_N_E - 1 - s_ref[b])
         for b in range(_B)], axis=1)
    p_s[0] = p0

    def body(t, carry):
        p, j = carry
        advrow = adv_s[pl.ds(t, 1)].reshape(_B, _W)
        a = jnp.sum(p * advrow, axis=1, keepdims=True)
        pshift = jnp.concatenate([p[:, :1] * 0.0, p[:, :-1]], axis=1)
        p = jnp.where(a > 0.0, pshift, p)
        j = j + a.astype(jnp.int32)
        p_s[pl.ds(t, 1)] = p.reshape(1, _B, _W)
        return (p, j)

    pT, jT = jax.lax.fori_loop(1, _T, body, (p0, j0))

    pall = p_s[...]
    jf = jnp.sum(pall * iot_j[None].astype(jnp.float32), axis=2)
    jint = jf.astype(jnp.int32)
    enc_ref[...] = jint
    v_ref[...] = jnp.max(jT - j0)[None, None]

    is_last = (jint == lim_row)
    proll = jnp.concatenate([pall[:, :, :1] * 0.0, pall[:, :, :-1]], axis=2)
    pnext = jnp.where(is_last[:, :, None], pall, proll)
    for b in range(_B):
        s = pl.multiple_of(s_ref[b], 256)
        wwin = w_ref[pl.ds(s, _W), :]
        kh_ref[b] = jax.lax.dot_general(
            pall[:, b, :], wwin, (((1,), (0,)), ((), ())),
            precision=jax.lax.Precision.HIGHEST,
            preferred_element_type=jnp.float32)
        kn_ref[b] = jax.lax.dot_general(
            pnext[:, b, :], wwin, (((1,), (0,)), ((), ())),
            precision=jax.lax.Precision.HIGHEST,
            preferred_element_type=jnp.float32)


def _walk_pass(sb, j0, key_soft, w_pad):
    return pl.pallas_call(
        _walk_kernel,
        grid=(1,),
        in_specs=[
            pl.BlockSpec(memory_space=pltpu.SMEM),
            pl.BlockSpec(memory_space=pltpu.SMEM),
            pl.BlockSpec((_B, _T, _E), lambda i: (0, 0, 0)),
            pl.BlockSpec((_NPAD, _E), lambda i: (0, 0)),
        ],
        out_specs=[
            pl.BlockSpec((_T, _B), lambda i: (0, 0)),
            pl.BlockSpec((1, 1), lambda i: (0, 0)),
            pl.BlockSpec((_B, _T, _E), lambda i: (0, 0, 0)),
            pl.BlockSpec((_B, _T, _E), lambda i: (0, 0, 0)),
        ],
        out_shape=[
            jax.ShapeDtypeStruct((_T, _B), jnp.int32),
            jax.ShapeDtypeStruct((1, 1), jnp.int32),
            jax.ShapeDtypeStruct((_B, _T, _E), jnp.float32),
            jax.ShapeDtypeStruct((_B, _T, _E), jnp.float32),
        ],
        scratch_shapes=[
            pltpu.VMEM((_T, _B, _W), jnp.float32),
            pltpu.VMEM((_T, _B, _W), jnp.float32),
            pltpu.VMEM((_T, _B, _W), jnp.float32),
        ],
    )(sb, j0, key_soft, w_pad)


def kernel(key_soft, W):
    Bx, Tx, e_dim = key_soft.shape
    n_e = W.shape[0] - 1

    ksf = key_soft.reshape(-1, e_dim)
    w_pad = jnp.concatenate(
        [W, jnp.full((_NPAD - (n_e + 1), e_dim), 30000.0, jnp.float32)], axis=0)

    min_val, min_idx = _argmin_pass(ksf, w_pad)
    min_indices = min_idx[:, 0]
    enc0 = jnp.clip(min_indices.reshape(Bx, Tx)[:, 0], 0, n_e - 1)
    sb = (enc0 // 256) * 256
    j0 = enc0 - sb

    enc_t, v11, key_hard_here, key_hard_next = _walk_pass(
        sb, j0, key_soft, w_pad)
    encoding_indices = enc_t.T + sb[:, None]
    v = v11.reshape(())

    key_min = jnp.take(W, min_indices, axis=0).reshape(key_soft.shape)

    lc = _LC
    loss_here_base = jnp.sum((key_soft - key_hard_here) ** 2, axis=-1) * lc + jnp.sum((key_soft - key_hard_here) ** 2, axis=-1)
    loss_next_base = jnp.sum((key_soft - key_hard_next) ** 2, axis=-1) * lc + jnp.sum((key_soft - key_hard_next) ** 2, axis=-1)
    loss_min_indices = jnp.sum((key_soft - key_min) ** 2, axis=-1) + jnp.sum((key_soft - key_min) ** 2, axis=-1) * lc
    loss_min_here = jnp.where(loss_min_indices < loss_here_base, loss_min_indices, 0.0)
    loss_min_next = jnp.where(loss_min_indices < loss_next_base, loss_min_indices, 0.0)
    loss_here = loss_here_base - loss_min_here
    loss_next = loss_next_base - loss_min_next

    key_hard = key_hard_here

    min_i = jnp.min(encoding_indices, axis=1)
    max_i = jnp.max(encoding_indices, axis=1)
    v_out = jnp.max(max_i - min_i)
    del v_out
    return (key_hard, encoding_indices, v, loss_here, loss_next)
